# baseline (speedup 1.0000x reference)
import functools
import math

import jax
import jax.numpy as jnp
from jax import lax
from jax.experimental import pallas as pl
from jax.experimental.pallas import tpu as pltpu
from jax.experimental.pallas import tpu_sc as plsc

D_MODEL = 64
NUM_HEADS = 8
SCALE = math.sqrt(D_MODEL)
ROW = NUM_HEADS * D_MODEL
BATCH, SEQ = 2, 128
B = BATCH * SEQ * SEQ
NC, NS = 2, 16
NW = NC * NS
B_PER_W = B // NW
CHUNK = 64
N_CHUNKS = B_PER_W // CHUNK


@functools.partial(
    pl.kernel,
    out_type=(
        jax.ShapeDtypeStruct((B, ROW), jnp.float32),
        jax.ShapeDtypeStruct((B, ROW), jnp.float32),
    ),
    mesh=plsc.VectorSubcoreMesh(core_axis_name="c", subcore_axis_name="s"),
    scratch_types=[
        pltpu.VMEM((CHUNK,), jnp.int32),
        pltpu.VMEM((CHUNK, ROW), jnp.float32),
        pltpu.VMEM((CHUNK, ROW), jnp.float32),
        pltpu.SemaphoreType.DMA,
        pltpu.SemaphoreType.DMA,
    ],
)
def _rel_pos_gather(ktab, vtab, idx, k_out, v_out, idx_v, kbuf, vbuf, ksem, vsem):
    wid = lax.axis_index("s") * NC + lax.axis_index("c")
    base = wid * B_PER_W
    for g in range(N_CHUNKS):
        off = base + g * CHUNK
        pltpu.sync_copy(idx.at[pl.ds(off, CHUNK)], idx_v)
        kcp = pltpu.async_copy(ktab.at[idx_v], kbuf, ksem)
        vcp = pltpu.async_copy(vtab.at[idx_v], vbuf, vsem)
        kcp.wait()
        vcp.wait()
        pltpu.sync_copy(kbuf, k_out.at[pl.ds(off, CHUNK)])
        pltpu.sync_copy(vbuf, v_out.at[pl.ds(off, CHUNK)])


def kernel(inputs, relation_type, parent_emb, brother_emb):
    if isinstance(relation_type, str) and relation_type == "parent":
        table = parent_emb
    else:
        table = brother_emb
    table = table.at[1].set(0.0) * SCALE
    ktab = jnp.tile(table[:, :D_MODEL], (1, NUM_HEADS))
    vtab = jnp.tile(table[:, D_MODEL:], (1, NUM_HEADS))
    idx = inputs.reshape(B)
    k_flat, v_flat = _rel_pos_gather(ktab, vtab, idx)
    out_shape = (BATCH * NUM_HEADS, SEQ, SEQ, D_MODEL)
    return (k_flat.reshape(out_shape), v_flat.reshape(out_shape))
